# baseline (speedup 1.0000x reference)
import functools

import jax
import jax.numpy as jnp
from jax import lax
from jax.experimental import pallas as pl
from jax.experimental.pallas import tpu as pltpu
from jax.experimental.pallas import tpu_sc as plsc

N = 10000
DIN = 128
C = 128
TILES = 32
RPS = 640
NP = RPS * 16
DEGW = 8
R = 2000
GRID = N // R

_mesh = plsc.VectorSubcoreMesh(core_axis_name="c", subcore_axis_name="s")


def _hist_body(dst2d, zfeat, out, didx, hist2, outrows, idv, shdeg, sem):
    cpt = dst2d.shape[0] // TILES
    nfr = NP // 128
    cid = lax.axis_index("c")
    sid = lax.axis_index("s")
    wid = cid * 16 + sid
    pltpu.sync_copy(zfeat.at[pl.ds(0, nfr)], hist2)

    @pl.when(sid == 0)
    def _zero_shared():
        pltpu.sync_copy(zfeat.at[pl.ds(0, nfr)], shdeg)

    pltpu.sync_copy(dst2d.at[pl.ds(wid * cpt, cpt)], didx)
    iota = lax.iota(jnp.int32, 16)
    for m in range(nfr // 16):
        idv[pl.ds(m * 16, 16)] = iota + m * 16
    ones = jnp.ones((16,), jnp.float32)

    def body(j, carry):
        for k in range(8):
            v = didx[j, pl.ds(k * 16, 16)]
            plsc.addupdate_scatter(hist2, [v >> 7, v & 127], ones)
        return carry

    lax.fori_loop(0, cpt, body, 0)
    plsc.subcore_barrier()
    pltpu.sync_copy(hist2, shdeg.at[idv], add=True)
    plsc.subcore_barrier()

    @pl.when(sid < 10)
    def _expand():
        pltpu.sync_copy(shdeg.at[pl.ds(sid * 8, 8)], hist2.at[pl.ds(0, 8)])
        zero = jnp.zeros((16,), jnp.int32)
        for r in range(8):
            for m in range(8):
                val = hist2[r, pl.ds(m * 16, 16)]
                plsc.store_scatter(outrows, [iota + m * 16, zero], val)
            pltpu.sync_copy(
                outrows, out.at[cid, pl.ds(sid * 1024 + r * 128, 128)])


def _prop_body(xs, src2d, dst2d, zfeat, out, sidx, didx,
               rows0, rows1, shacc, sg0, sg1, ss0, ss1):
    cpt = src2d.shape[0] // TILES
    half = cpt // 2
    npair = half // 2
    cid = lax.axis_index("c")
    sid = lax.axis_index("s")
    wid = cid * 16 + sid
    rbase = sid * RPS
    pltpu.sync_copy(zfeat.at[pl.ds(rbase, RPS)], shacc.at[pl.ds(rbase, RPS)])
    plsc.subcore_barrier()

    for h in range(2):
        pltpu.sync_copy(src2d.at[pl.ds((wid * 2 + h) * half, half)], sidx)
        pltpu.sync_copy(dst2d.at[pl.ds((wid * 2 + h) * half, half)], didx)
        pltpu.async_copy(xs.at[sidx.at[0]], rows0, sg0)
        pltpu.async_copy(xs.at[sidx.at[1]], rows1, sg1)

        def pair(k, carry):
            j0 = k * 2
            j1 = j0 + 1
            pltpu.make_async_copy(xs.at[sidx.at[j0]], rows0, sg0).wait()
            pltpu.async_copy(rows0, shacc.at[didx.at[j0]], ss0, add=True)
            pltpu.make_async_copy(xs.at[sidx.at[j1]], rows1, sg1).wait()
            pltpu.async_copy(rows1, shacc.at[didx.at[j1]], ss1, add=True)
            pltpu.make_async_copy(rows0, shacc.at[didx.at[j0]], ss0).wait()
            pltpu.make_async_copy(rows1, shacc.at[didx.at[j1]], ss1).wait()

            @pl.when(k < npair - 1)
            def _():
                pltpu.async_copy(xs.at[sidx.at[j0 + 2]], rows0, sg0)
                pltpu.async_copy(xs.at[sidx.at[j1 + 2]], rows1, sg1)

            return carry

        lax.fori_loop(0, npair, pair, 0)

    plsc.subcore_barrier()
    pltpu.sync_copy(shacc.at[pl.ds(rbase, RPS)], out.at[cid, pl.ds(rbase, RPS)])


def _sc_hist(dst2d, zfeat):
    cpt = dst2d.shape[0] // TILES
    fn = functools.partial(
        pl.kernel,
        mesh=_mesh,
        out_type=jax.ShapeDtypeStruct((2, NP, 8), jnp.float32),
        scratch_types=[
            pltpu.VMEM((cpt, C), jnp.int32),
            pltpu.VMEM((NP // 128, 128), jnp.float32),
            pltpu.VMEM((128, 8), jnp.float32),
            pltpu.VMEM((NP // 128,), jnp.int32),
            pltpu.VMEM_SHARED((NP // 128, 128), jnp.float32),
            pltpu.SemaphoreType.DMA,
        ],
        compiler_params=pltpu.CompilerParams(needs_layout_passes=False),
    )(_hist_body)
    return fn(dst2d, zfeat)


def _sc_prop(xs, src2d, dst2d, zfeat):
    cpt = src2d.shape[0] // TILES
    fn = functools.partial(
        pl.kernel,
        mesh=_mesh,
        out_type=jax.ShapeDtypeStruct((2, NP, DIN), jnp.float32),
        scratch_types=[
            pltpu.VMEM((cpt // 2, C), jnp.int32),
            pltpu.VMEM((cpt // 2, C), jnp.int32),
            pltpu.VMEM((C, DIN), jnp.float32),
            pltpu.VMEM((C, DIN), jnp.float32),
            pltpu.VMEM_SHARED((NP, DIN), jnp.float32),
            pltpu.SemaphoreType.DMA,
            pltpu.SemaphoreType.DMA,
            pltpu.SemaphoreType.DMA,
            pltpu.SemaphoreType.DMA,
        ],
    )(_prop_body)
    return fn(xs, src2d, dst2d, zfeat)


def _prep_body(deg_ref, x_ref, xs_ref, dinv_ref):
    deg = deg_ref[0, :, 0:1] + deg_ref[1, :, 0:1] + 1.0
    dinv = lax.rsqrt(deg)
    dinv_ref[...] = jnp.broadcast_to(dinv, dinv_ref.shape)
    xs_ref[...] = x_ref[...] * dinv


def _tc_prep(deg, x):
    return pl.pallas_call(
        _prep_body,
        grid=(GRID,),
        in_specs=[
            pl.BlockSpec((2, R, 8), lambda r: (0, r, 0)),
            pl.BlockSpec((R, DIN), lambda r: (r, 0)),
        ],
        out_specs=[
            pl.BlockSpec((R, DIN), lambda r: (r, 0)),
            pl.BlockSpec((R, DEGW), lambda r: (r, 0)),
        ],
        out_shape=[
            jax.ShapeDtypeStruct((NP, DIN), jnp.float32),
            jax.ShapeDtypeStruct((N, DEGW), jnp.float32),
        ],
    )(deg, x)


def _mid_body(acc_ref, xs1_ref, dinv_ref, w1_ref, b1_ref, w2_ref, xs2_ref):
    dinv = dinv_ref[:, 0:1]
    s1 = (acc_ref[0] + acc_ref[1] + xs1_ref[...]) * dinv
    f = jnp.maximum(
        jnp.dot(s1, w1_ref[...], preferred_element_type=jnp.float32) + b1_ref[...],
        0.0)
    xw2 = jnp.dot(f, w2_ref[...], preferred_element_type=jnp.float32)
    xs2_ref[...] = xw2 * dinv


def _tc_mid(acc, xs1, dinv, W1, b1, W2):
    return pl.pallas_call(
        _mid_body,
        grid=(GRID,),
        in_specs=[
            pl.BlockSpec((2, R, DIN), lambda r: (0, r, 0)),
            pl.BlockSpec((R, DIN), lambda r: (r, 0)),
            pl.BlockSpec((R, DEGW), lambda r: (r, 0)),
            pl.BlockSpec(W1.shape, lambda r: (0, 0)),
            pl.BlockSpec((1, 256), lambda r: (0, 0)),
            pl.BlockSpec(W2.shape, lambda r: (0, 0)),
        ],
        out_specs=pl.BlockSpec((R, DIN), lambda r: (r, 0)),
        out_shape=jax.ShapeDtypeStruct((NP, DIN), jnp.float32),
    )(acc, xs1, dinv, W1, b1, W2)


def _fin_body(acc_ref, xs2_ref, dinv_ref, x_ref, b2_ref,
              wgz_ref, wgx_ref, bg_ref, wo_ref, bo_ref,
              h_ref, out_ref, zsum_ref):
    r = pl.program_id(0)
    dinv = dinv_ref[:, 0:1]
    z = jnp.maximum(
        (acc_ref[0] + acc_ref[1] + xs2_ref[...]) * dinv + b2_ref[...], 0.0)
    go = (jnp.dot(z, wgz_ref[...], preferred_element_type=jnp.float32)
          + jnp.dot(x_ref[...], wgx_ref[...], preferred_element_type=jnp.float32)
          + bg_ref[...])
    g = jax.nn.sigmoid(go[:, :256])
    ht = jnp.tanh(go[:, 256:])
    h_ref[...] = (1.0 - g) * ht

    @pl.when(r == 0)
    def _init():
        zsum_ref[...] = jnp.zeros_like(zsum_ref)

    zsum_ref[...] += jnp.sum(z, axis=0, keepdims=True)

    @pl.when(r == GRID - 1)
    def _head():
        out_ref[...] = (
            jnp.dot(zsum_ref[...] * (1.0 / N), wo_ref[...],
                    preferred_element_type=jnp.float32) + bo_ref[...])


def _tc_fin(acc, xs2, dinv, x, b2, Wgz, Wgx, bg, Wo, bo):
    return pl.pallas_call(
        _fin_body,
        grid=(GRID,),
        in_specs=[
            pl.BlockSpec((2, R, DIN), lambda r: (0, r, 0)),
            pl.BlockSpec((R, DIN), lambda r: (r, 0)),
            pl.BlockSpec((R, DEGW), lambda r: (r, 0)),
            pl.BlockSpec((R, DIN), lambda r: (r, 0)),
            pl.BlockSpec((1, DIN), lambda r: (0, 0)),
            pl.BlockSpec(Wgz.shape, lambda r: (0, 0)),
            pl.BlockSpec(Wgx.shape, lambda r: (0, 0)),
            pl.BlockSpec((1, 512), lambda r: (0, 0)),
            pl.BlockSpec(Wo.shape, lambda r: (0, 0)),
            pl.BlockSpec((1, 1), lambda r: (0, 0)),
        ],
        out_specs=[
            pl.BlockSpec((R, 256), lambda r: (r, 0)),
            pl.BlockSpec((1, 1), lambda r: (0, 0)),
            pl.BlockSpec((1, DIN), lambda r: (0, 0)),
        ],
        out_shape=[
            jax.ShapeDtypeStruct((N, 256), jnp.float32),
            jax.ShapeDtypeStruct((1, 1), jnp.float32),
            jax.ShapeDtypeStruct((1, DIN), jnp.float32),
        ],
    )(acc, xs2, dinv, x, b2, Wgz, Wgx, bg, Wo, bo)


def kernel(x, edge_index, W1, b1, W2, b2, Wxz, bxz, Whz, bhz, Wxr, bxr,
           Whr, bhr, Wxh, bxh, Whh, bhh, Wo, bo):
    e = edge_index.shape[1]
    cpt = -(-e // (TILES * C))
    cpt = (cpt + 7) // 8 * 8
    ep = TILES * cpt * C
    eidx = edge_index.astype(jnp.int32).reshape(2, e // C, C)
    pad_i = jnp.arange(ep - e, dtype=jnp.int32)
    pad_src = (pad_i % 256).reshape(-1, C)
    pad_dst = (N + pad_i % (NP - N)).reshape(-1, C)
    src2d = jnp.concatenate([eidx[0], pad_src], axis=0)
    dst2d = jnp.concatenate([eidx[1], pad_dst], axis=0)

    zfeat = jnp.zeros((NP, DIN), jnp.float32)

    deg = _sc_hist(dst2d, zfeat)
    xs1, dinv = _tc_prep(deg, x)
    acc1 = _sc_prop(xs1, src2d, dst2d, zfeat)
    xs2 = _tc_mid(acc1, xs1, dinv, W1, b1.reshape(1, 256), W2)
    acc2 = _sc_prop(xs2, src2d, dst2d, zfeat)
    Wgz = jnp.concatenate([Wxz[:DIN], Wxh[:DIN]], axis=1)
    Wgx = jnp.concatenate([Wxz[DIN:], Wxh[DIN:]], axis=1)
    bg = jnp.concatenate([bxz + bhz, bxh + bhh]).reshape(1, 512)
    H, out, _ = _tc_fin(
        acc2, xs2, dinv, x, b2.reshape(1, DIN), Wgz, Wgx, bg,
        Wo, bo.reshape(1, 1))
    return (out.reshape(1), H)


# speedup vs baseline: 1.2495x; 1.2495x over previous
import functools

import jax
import jax.numpy as jnp
from jax import lax
from jax.experimental import pallas as pl
from jax.experimental.pallas import tpu as pltpu
from jax.experimental.pallas import tpu_sc as plsc

N = 10000
DIN = 128
C = 128
TILES = 32
RPS = 640
NP = RPS * 16
DEGW = 8
R = 2000
GRID = N // R

_mesh = plsc.VectorSubcoreMesh(core_axis_name="c", subcore_axis_name="s")


def _hist_body(dst2d, zfeat, out, didx, hist2, outrows, idv, shdeg, sem):
    cpt = dst2d.shape[0] // TILES
    nfr = NP // 128
    cid = lax.axis_index("c")
    sid = lax.axis_index("s")
    wid = cid * 16 + sid
    pltpu.sync_copy(zfeat.at[pl.ds(0, nfr)], hist2)

    @pl.when(sid == 0)
    def _zero_shared():
        pltpu.sync_copy(zfeat.at[pl.ds(0, nfr)], shdeg)

    pltpu.sync_copy(dst2d.at[pl.ds(wid * cpt, cpt)], didx)
    iota = lax.iota(jnp.int32, 16)
    for m in range(nfr // 16):
        idv[pl.ds(m * 16, 16)] = iota + m * 16
    ones = jnp.ones((16,), jnp.float32)

    def body(j, carry):
        for k in range(8):
            v = didx[j, pl.ds(k * 16, 16)]
            plsc.addupdate_scatter(hist2, [v >> 7, v & 127], ones)
        return carry

    lax.fori_loop(0, cpt, body, 0)
    plsc.subcore_barrier()
    pltpu.sync_copy(hist2, shdeg.at[idv], add=True)
    plsc.subcore_barrier()

    @pl.when(sid < 10)
    def _expand():
        pltpu.sync_copy(shdeg.at[pl.ds(sid * 8, 8)], hist2.at[pl.ds(0, 8)])
        zero = jnp.zeros((16,), jnp.int32)
        for r in range(8):
            for m in range(8):
                val = hist2[r, pl.ds(m * 16, 16)]
                plsc.store_scatter(outrows, [iota + m * 16, zero], val)
            pltpu.sync_copy(
                outrows, out.at[cid, pl.ds(sid * 1024 + r * 128, 128)])


def _prop_body(xs, src2d, dst2d, zfeat, out, sidx, didx,
               rows0, rows1, shacc, sg0, sg1, ss0, ss1):
    cpt = src2d.shape[0] // TILES
    half = cpt // 2
    npair = half // 2
    cid = lax.axis_index("c")
    sid = lax.axis_index("s")
    wid = cid * 16 + sid
    rbase = sid * RPS
    pltpu.sync_copy(zfeat.at[pl.ds(rbase, RPS)], shacc.at[pl.ds(rbase, RPS)])
    plsc.subcore_barrier()

    for h in range(2):
        pltpu.sync_copy(src2d.at[pl.ds((wid * 2 + h) * half, half)], sidx)
        pltpu.sync_copy(dst2d.at[pl.ds((wid * 2 + h) * half, half)], didx)
        pltpu.async_copy(xs.at[sidx.at[0]], rows0, sg0)

        def pair(k, carry):
            j0 = k * 2
            j1 = j0 + 1

            @pl.when(k > 0)
            def _():
                pltpu.make_async_copy(
                    rows1, shacc.at[didx.at[j0 - 1]], ss1).wait()

            pltpu.async_copy(xs.at[sidx.at[j1]], rows1, sg1)
            pltpu.make_async_copy(xs.at[sidx.at[j0]], rows0, sg0).wait()
            pltpu.async_copy(rows0, shacc.at[didx.at[j0]], ss0, add=True)

            pltpu.make_async_copy(rows0, shacc.at[didx.at[j0]], ss0).wait()

            @pl.when(k < npair - 1)
            def _():
                pltpu.async_copy(xs.at[sidx.at[j0 + 2]], rows0, sg0)

            pltpu.make_async_copy(xs.at[sidx.at[j1]], rows1, sg1).wait()
            pltpu.async_copy(rows1, shacc.at[didx.at[j1]], ss1, add=True)
            return carry

        lax.fori_loop(0, npair, pair, 0)
        pltpu.make_async_copy(rows1, shacc.at[didx.at[half - 1]], ss1).wait()

    plsc.subcore_barrier()
    pltpu.sync_copy(shacc.at[pl.ds(rbase, RPS)], out.at[cid, pl.ds(rbase, RPS)])


def _sc_hist(dst2d, zfeat):
    cpt = dst2d.shape[0] // TILES
    fn = functools.partial(
        pl.kernel,
        mesh=_mesh,
        out_type=jax.ShapeDtypeStruct((2, NP, 8), jnp.float32),
        scratch_types=[
            pltpu.VMEM((cpt, C), jnp.int32),
            pltpu.VMEM((NP // 128, 128), jnp.float32),
            pltpu.VMEM((128, 8), jnp.float32),
            pltpu.VMEM((NP // 128,), jnp.int32),
            pltpu.VMEM_SHARED((NP // 128, 128), jnp.float32),
            pltpu.SemaphoreType.DMA,
        ],
        compiler_params=pltpu.CompilerParams(needs_layout_passes=False),
    )(_hist_body)
    return fn(dst2d, zfeat)


def _sc_prop(xs, src2d, dst2d, zfeat):
    cpt = src2d.shape[0] // TILES
    fn = functools.partial(
        pl.kernel,
        mesh=_mesh,
        out_type=jax.ShapeDtypeStruct((2, NP, DIN), jnp.float32),
        scratch_types=[
            pltpu.VMEM((cpt // 2, C), jnp.int32),
            pltpu.VMEM((cpt // 2, C), jnp.int32),
            pltpu.VMEM((C, DIN), jnp.float32),
            pltpu.VMEM((C, DIN), jnp.float32),
            pltpu.VMEM_SHARED((NP, DIN), jnp.float32),
            pltpu.SemaphoreType.DMA,
            pltpu.SemaphoreType.DMA,
            pltpu.SemaphoreType.DMA,
            pltpu.SemaphoreType.DMA,
        ],
    )(_prop_body)
    return fn(xs, src2d, dst2d, zfeat)


def _prep_body(deg_ref, x_ref, xs_ref, dinv_ref):
    deg = deg_ref[0, :, 0:1] + deg_ref[1, :, 0:1] + 1.0
    dinv = lax.rsqrt(deg)
    dinv_ref[...] = jnp.broadcast_to(dinv, dinv_ref.shape)
    xs_ref[...] = x_ref[...] * dinv


def _tc_prep(deg, x):
    return pl.pallas_call(
        _prep_body,
        grid=(GRID,),
        in_specs=[
            pl.BlockSpec((2, R, 8), lambda r: (0, r, 0)),
            pl.BlockSpec((R, DIN), lambda r: (r, 0)),
        ],
        out_specs=[
            pl.BlockSpec((R, DIN), lambda r: (r, 0)),
            pl.BlockSpec((R, DEGW), lambda r: (r, 0)),
        ],
        out_shape=[
            jax.ShapeDtypeStruct((NP, DIN), jnp.float32),
            jax.ShapeDtypeStruct((N, DEGW), jnp.float32),
        ],
    )(deg, x)


def _mid_body(acc_ref, xs1_ref, dinv_ref, w1_ref, b1_ref, w2_ref, xs2_ref):
    dinv = dinv_ref[:, 0:1]
    s1 = (acc_ref[0] + acc_ref[1] + xs1_ref[...]) * dinv
    f = jnp.maximum(
        jnp.dot(s1, w1_ref[...], preferred_element_type=jnp.float32) + b1_ref[...],
        0.0)
    xw2 = jnp.dot(f, w2_ref[...], preferred_element_type=jnp.float32)
    xs2_ref[...] = xw2 * dinv


def _tc_mid(acc, xs1, dinv, W1, b1, W2):
    return pl.pallas_call(
        _mid_body,
        grid=(GRID,),
        in_specs=[
            pl.BlockSpec((2, R, DIN), lambda r: (0, r, 0)),
            pl.BlockSpec((R, DIN), lambda r: (r, 0)),
            pl.BlockSpec((R, DEGW), lambda r: (r, 0)),
            pl.BlockSpec(W1.shape, lambda r: (0, 0)),
            pl.BlockSpec((1, 256), lambda r: (0, 0)),
            pl.BlockSpec(W2.shape, lambda r: (0, 0)),
        ],
        out_specs=pl.BlockSpec((R, DIN), lambda r: (r, 0)),
        out_shape=jax.ShapeDtypeStruct((NP, DIN), jnp.float32),
    )(acc, xs1, dinv, W1, b1, W2)


def _fin_body(acc_ref, xs2_ref, dinv_ref, x_ref, b2_ref,
              wgz_ref, wgx_ref, bg_ref, wo_ref, bo_ref,
              h_ref, out_ref, zsum_ref):
    r = pl.program_id(0)
    dinv = dinv_ref[:, 0:1]
    z = jnp.maximum(
        (acc_ref[0] + acc_ref[1] + xs2_ref[...]) * dinv + b2_ref[...], 0.0)
    go = (jnp.dot(z, wgz_ref[...], preferred_element_type=jnp.float32)
          + jnp.dot(x_ref[...], wgx_ref[...], preferred_element_type=jnp.float32)
          + bg_ref[...])
    g = jax.nn.sigmoid(go[:, :256])
    ht = jnp.tanh(go[:, 256:])
    h_ref[...] = (1.0 - g) * ht

    @pl.when(r == 0)
    def _init():
        zsum_ref[...] = jnp.zeros_like(zsum_ref)

    zsum_ref[...] += jnp.sum(z, axis=0, keepdims=True)

    @pl.when(r == GRID - 1)
    def _head():
        out_ref[...] = (
            jnp.dot(zsum_ref[...] * (1.0 / N), wo_ref[...],
                    preferred_element_type=jnp.float32) + bo_ref[...])


def _tc_fin(acc, xs2, dinv, x, b2, Wgz, Wgx, bg, Wo, bo):
    return pl.pallas_call(
        _fin_body,
        grid=(GRID,),
        in_specs=[
            pl.BlockSpec((2, R, DIN), lambda r: (0, r, 0)),
            pl.BlockSpec((R, DIN), lambda r: (r, 0)),
            pl.BlockSpec((R, DEGW), lambda r: (r, 0)),
            pl.BlockSpec((R, DIN), lambda r: (r, 0)),
            pl.BlockSpec((1, DIN), lambda r: (0, 0)),
            pl.BlockSpec(Wgz.shape, lambda r: (0, 0)),
            pl.BlockSpec(Wgx.shape, lambda r: (0, 0)),
            pl.BlockSpec((1, 512), lambda r: (0, 0)),
            pl.BlockSpec(Wo.shape, lambda r: (0, 0)),
            pl.BlockSpec((1, 1), lambda r: (0, 0)),
        ],
        out_specs=[
            pl.BlockSpec((R, 256), lambda r: (r, 0)),
            pl.BlockSpec((1, 1), lambda r: (0, 0)),
            pl.BlockSpec((1, DIN), lambda r: (0, 0)),
        ],
        out_shape=[
            jax.ShapeDtypeStruct((N, 256), jnp.float32),
            jax.ShapeDtypeStruct((1, 1), jnp.float32),
            jax.ShapeDtypeStruct((1, DIN), jnp.float32),
        ],
    )(acc, xs2, dinv, x, b2, Wgz, Wgx, bg, Wo, bo)


def kernel(x, edge_index, W1, b1, W2, b2, Wxz, bxz, Whz, bhz, Wxr, bxr,
           Whr, bhr, Wxh, bxh, Whh, bhh, Wo, bo):
    e = edge_index.shape[1]
    cpt = -(-e // (TILES * C))
    cpt = (cpt + 7) // 8 * 8
    ep = TILES * cpt * C
    eidx = edge_index.astype(jnp.int32).reshape(2, e // C, C)
    pad_i = jnp.arange(ep - e, dtype=jnp.int32)
    pad_src = (pad_i % 256).reshape(-1, C)
    pad_dst = (N + pad_i % (NP - N)).reshape(-1, C)
    src2d = jnp.concatenate([eidx[0], pad_src], axis=0)
    dst2d = jnp.concatenate([eidx[1], pad_dst], axis=0)

    zfeat = jnp.zeros((NP, DIN), jnp.float32)

    deg = _sc_hist(dst2d, zfeat)
    xs1, dinv = _tc_prep(deg, x)
    acc1 = _sc_prop(xs1, src2d, dst2d, zfeat)
    xs2 = _tc_mid(acc1, xs1, dinv, W1, b1.reshape(1, 256), W2)
    acc2 = _sc_prop(xs2, src2d, dst2d, zfeat)
    Wgz = jnp.concatenate([Wxz[:DIN], Wxh[:DIN]], axis=1)
    Wgx = jnp.concatenate([Wxz[DIN:], Wxh[DIN:]], axis=1)
    bg = jnp.concatenate([bxz + bhz, bxh + bhh]).reshape(1, 512)
    H, out, _ = _tc_fin(
        acc2, xs2, dinv, x, b2.reshape(1, DIN), Wgz, Wgx, bg,
        Wo, bo.reshape(1, 1))
    return (out.reshape(1), H)


# speedup vs baseline: 1.3078x; 1.0466x over previous
import functools

import jax
import jax.numpy as jnp
from jax import lax
from jax.experimental import pallas as pl
from jax.experimental.pallas import tpu as pltpu
from jax.experimental.pallas import tpu_sc as plsc

N = 10000
DIN = 128
C = 128
TILES = 32
RPS = 640
NP = RPS * 16
DEGW = 8
R = 2000
GRID = N // R

_mesh = plsc.VectorSubcoreMesh(core_axis_name="c", subcore_axis_name="s")


def _hist_body(eidx3, zfeat, out, didx, hist2, outrows, idv, shdeg, sem):
    full = eidx3.shape[1]
    cpt = -(-full // TILES)
    cpt = (cpt + 7) // 8 * 8
    tail = full - (TILES - 1) * cpt
    nfr = NP // 128
    cid = lax.axis_index("c")
    sid = lax.axis_index("s")
    wid = cid * 16 + sid
    pltpu.sync_copy(zfeat.at[pl.ds(0, nfr)], hist2)

    @pl.when(sid == 0)
    def _zero_shared():
        pltpu.sync_copy(zfeat.at[pl.ds(0, nfr)], shdeg)

    @pl.when(wid < TILES - 1)
    def _load_full():
        pltpu.sync_copy(eidx3.at[1, pl.ds(wid * cpt, cpt)], didx)

    @pl.when(wid == TILES - 1)
    def _load_tail():
        pltpu.sync_copy(eidx3.at[1, pl.ds((TILES - 1) * cpt, tail)],
                        didx.at[pl.ds(0, tail)])

    cnt = jnp.where(wid < TILES - 1, cpt, tail)
    iota = lax.iota(jnp.int32, 16)
    for m in range(nfr // 16):
        idv[pl.ds(m * 16, 16)] = iota + m * 16
    ones = jnp.ones((16,), jnp.float32)

    def body(j, carry):
        for k in range(8):
            v = didx[j, pl.ds(k * 16, 16)]
            plsc.addupdate_scatter(hist2, [v >> 7, v & 127], ones)
        return carry

    lax.fori_loop(0, cnt, body, 0)
    plsc.subcore_barrier()
    pltpu.sync_copy(hist2, shdeg.at[idv], add=True)
    plsc.subcore_barrier()

    @pl.when(sid < 10)
    def _expand():
        pltpu.sync_copy(shdeg.at[pl.ds(sid * 8, 8)], hist2.at[pl.ds(0, 8)])
        zero = jnp.zeros((16,), jnp.int32)
        for r in range(8):
            for m in range(8):
                val = hist2[r, pl.ds(m * 16, 16)]
                plsc.store_scatter(outrows, [iota + m * 16, zero], val)
            pltpu.sync_copy(
                outrows, out.at[cid, pl.ds(sid * 1024 + r * 128, 128)])


def _prop_body(xs, eidx3, zfeat, out, sidx, didx,
               rows0, rows1, shacc, sg0, sg1, ss0, ss1):
    full = eidx3.shape[1]
    cpt = -(-full // TILES)
    cpt = (cpt + 7) // 8 * 8
    tail = full - (TILES - 1) * cpt
    half = cpt // 2
    cid = lax.axis_index("c")
    sid = lax.axis_index("s")
    wid = cid * 16 + sid
    rbase = sid * RPS
    pltpu.sync_copy(zfeat.at[pl.ds(rbase, RPS)], shacc.at[pl.ds(rbase, RPS)])
    plsc.subcore_barrier()

    for h in range(2):
        @pl.when(wid < TILES - 1)
        def _load_full():
            base = (wid * 2 + h) * half
            pltpu.sync_copy(eidx3.at[0, pl.ds(base, half)], sidx)
            pltpu.sync_copy(eidx3.at[1, pl.ds(base, half)], didx)

        if h == 0:
            @pl.when(wid == TILES - 1)
            def _load_tail():
                base = (TILES - 1) * cpt
                pltpu.sync_copy(eidx3.at[0, pl.ds(base, tail)],
                                sidx.at[pl.ds(0, tail)])
                pltpu.sync_copy(eidx3.at[1, pl.ds(base, tail)],
                                didx.at[pl.ds(0, tail)])

        hcnt = jnp.where(wid < TILES - 1, half, tail if h == 0 else 0)
        npair = hcnt // 2

        @pl.when(npair > 0)
        def _prologue():
            pltpu.async_copy(xs.at[sidx.at[0]], rows0, sg0)

        def pair(k, carry):
            j0 = k * 2
            j1 = j0 + 1

            @pl.when(k > 0)
            def _():
                pltpu.make_async_copy(
                    rows1, shacc.at[didx.at[j0 - 1]], ss1).wait()

            pltpu.async_copy(xs.at[sidx.at[j1]], rows1, sg1)
            pltpu.make_async_copy(xs.at[sidx.at[j0]], rows0, sg0).wait()
            pltpu.async_copy(rows0, shacc.at[didx.at[j0]], ss0, add=True)

            pltpu.make_async_copy(rows0, shacc.at[didx.at[j0]], ss0).wait()

            @pl.when(k < npair - 1)
            def _():
                pltpu.async_copy(xs.at[sidx.at[j0 + 2]], rows0, sg0)

            pltpu.make_async_copy(xs.at[sidx.at[j1]], rows1, sg1).wait()
            pltpu.async_copy(rows1, shacc.at[didx.at[j1]], ss1, add=True)
            return carry

        lax.fori_loop(0, npair, pair, 0)

        @pl.when(npair > 0)
        def _epilogue():
            pltpu.make_async_copy(
                rows1, shacc.at[didx.at[npair * 2 - 1]], ss1).wait()

    plsc.subcore_barrier()
    pltpu.sync_copy(shacc.at[pl.ds(rbase, RPS)], out.at[cid, pl.ds(rbase, RPS)])


def _sc_hist(eidx3, zfeat):
    full = eidx3.shape[1]
    cpt = (-(-full // TILES) + 7) // 8 * 8
    fn = functools.partial(
        pl.kernel,
        mesh=_mesh,
        out_type=jax.ShapeDtypeStruct((2, NP, 8), jnp.float32),
        scratch_types=[
            pltpu.VMEM((cpt, C), jnp.int32),
            pltpu.VMEM((NP // 128, 128), jnp.float32),
            pltpu.VMEM((128, 8), jnp.float32),
            pltpu.VMEM((NP // 128,), jnp.int32),
            pltpu.VMEM_SHARED((NP // 128, 128), jnp.float32),
            pltpu.SemaphoreType.DMA,
        ],
        compiler_params=pltpu.CompilerParams(needs_layout_passes=False),
    )(_hist_body)
    return fn(eidx3, zfeat)


def _sc_prop(xs, eidx3, zfeat):
    full = eidx3.shape[1]
    cpt = (-(-full // TILES) + 7) // 8 * 8
    fn = functools.partial(
        pl.kernel,
        mesh=_mesh,
        out_type=jax.ShapeDtypeStruct((2, NP, DIN), jnp.float32),
        scratch_types=[
            pltpu.VMEM((cpt // 2, C), jnp.int32),
            pltpu.VMEM((cpt // 2, C), jnp.int32),
            pltpu.VMEM((C, DIN), jnp.float32),
            pltpu.VMEM((C, DIN), jnp.float32),
            pltpu.VMEM_SHARED((NP, DIN), jnp.float32),
            pltpu.SemaphoreType.DMA,
            pltpu.SemaphoreType.DMA,
            pltpu.SemaphoreType.DMA,
            pltpu.SemaphoreType.DMA,
        ],
    )(_prop_body)
    return fn(xs, eidx3, zfeat)


def _prep_body(deg_ref, x_ref, xs_ref, dinv_ref):
    deg = deg_ref[0, :, 0:1] + deg_ref[1, :, 0:1] + 1.0
    dinv = lax.rsqrt(deg)
    dinv_ref[...] = jnp.broadcast_to(dinv, dinv_ref.shape)
    xs_ref[...] = x_ref[...] * dinv


def _tc_prep(deg, x):
    return pl.pallas_call(
        _prep_body,
        grid=(GRID,),
        in_specs=[
            pl.BlockSpec((2, R, 8), lambda r: (0, r, 0)),
            pl.BlockSpec((R, DIN), lambda r: (r, 0)),
        ],
        out_specs=[
            pl.BlockSpec((R, DIN), lambda r: (r, 0)),
            pl.BlockSpec((R, DEGW), lambda r: (r, 0)),
        ],
        out_shape=[
            jax.ShapeDtypeStruct((NP, DIN), jnp.float32),
            jax.ShapeDtypeStruct((N, DEGW), jnp.float32),
        ],
    )(deg, x)


def _mid_body(acc_ref, xs1_ref, dinv_ref, w1_ref, b1_ref, w2_ref, xs2_ref):
    dinv = dinv_ref[:, 0:1]
    s1 = (acc_ref[0] + acc_ref[1] + xs1_ref[...]) * dinv
    f = jnp.maximum(
        jnp.dot(s1, w1_ref[...], preferred_element_type=jnp.float32) + b1_ref[...],
        0.0)
    xw2 = jnp.dot(f, w2_ref[...], preferred_element_type=jnp.float32)
    xs2_ref[...] = xw2 * dinv


def _tc_mid(acc, xs1, dinv, W1, b1, W2):
    return pl.pallas_call(
        _mid_body,
        grid=(GRID,),
        in_specs=[
            pl.BlockSpec((2, R, DIN), lambda r: (0, r, 0)),
            pl.BlockSpec((R, DIN), lambda r: (r, 0)),
            pl.BlockSpec((R, DEGW), lambda r: (r, 0)),
            pl.BlockSpec(W1.shape, lambda r: (0, 0)),
            pl.BlockSpec((1, 256), lambda r: (0, 0)),
            pl.BlockSpec(W2.shape, lambda r: (0, 0)),
        ],
        out_specs=pl.BlockSpec((R, DIN), lambda r: (r, 0)),
        out_shape=jax.ShapeDtypeStruct((NP, DIN), jnp.float32),
    )(acc, xs1, dinv, W1, b1, W2)


def _fin_body(acc_ref, xs2_ref, dinv_ref, x_ref, b2_ref,
              wgz_ref, wgx_ref, bg_ref, wo_ref, bo_ref,
              h_ref, out_ref, zsum_ref):
    r = pl.program_id(0)
    dinv = dinv_ref[:, 0:1]
    z = jnp.maximum(
        (acc_ref[0] + acc_ref[1] + xs2_ref[...]) * dinv + b2_ref[...], 0.0)
    go = (jnp.dot(z, wgz_ref[...], preferred_element_type=jnp.float32)
          + jnp.dot(x_ref[...], wgx_ref[...], preferred_element_type=jnp.float32)
          + bg_ref[...])
    g = jax.nn.sigmoid(go[:, :256])
    ht = jnp.tanh(go[:, 256:])
    h_ref[...] = (1.0 - g) * ht

    @pl.when(r == 0)
    def _init():
        zsum_ref[...] = jnp.zeros_like(zsum_ref)

    zsum_ref[...] += jnp.sum(z, axis=0, keepdims=True)

    @pl.when(r == GRID - 1)
    def _head():
        out_ref[...] = (
            jnp.dot(zsum_ref[...] * (1.0 / N), wo_ref[...],
                    preferred_element_type=jnp.float32) + bo_ref[...])


def _tc_fin(acc, xs2, dinv, x, b2, Wgz, Wgx, bg, Wo, bo):
    return pl.pallas_call(
        _fin_body,
        grid=(GRID,),
        in_specs=[
            pl.BlockSpec((2, R, DIN), lambda r: (0, r, 0)),
            pl.BlockSpec((R, DIN), lambda r: (r, 0)),
            pl.BlockSpec((R, DEGW), lambda r: (r, 0)),
            pl.BlockSpec((R, DIN), lambda r: (r, 0)),
            pl.BlockSpec((1, DIN), lambda r: (0, 0)),
            pl.BlockSpec(Wgz.shape, lambda r: (0, 0)),
            pl.BlockSpec(Wgx.shape, lambda r: (0, 0)),
            pl.BlockSpec((1, 512), lambda r: (0, 0)),
            pl.BlockSpec(Wo.shape, lambda r: (0, 0)),
            pl.BlockSpec((1, 1), lambda r: (0, 0)),
        ],
        out_specs=[
            pl.BlockSpec((R, 256), lambda r: (r, 0)),
            pl.BlockSpec((1, 1), lambda r: (0, 0)),
            pl.BlockSpec((1, DIN), lambda r: (0, 0)),
        ],
        out_shape=[
            jax.ShapeDtypeStruct((N, 256), jnp.float32),
            jax.ShapeDtypeStruct((1, 1), jnp.float32),
            jax.ShapeDtypeStruct((1, DIN), jnp.float32),
        ],
    )(acc, xs2, dinv, x, b2, Wgz, Wgx, bg, Wo, bo)


def kernel(x, edge_index, W1, b1, W2, b2, Wxz, bxz, Whz, bhz, Wxr, bxr,
           Whr, bhr, Wxh, bxh, Whh, bhh, Wo, bo):
    e = edge_index.shape[1]
    eidx3 = edge_index.astype(jnp.int32).reshape(2, e // C, C)
    zfeat = jnp.zeros((NP, DIN), jnp.float32)

    deg = _sc_hist(eidx3, zfeat)
    xs1, dinv = _tc_prep(deg, x)
    acc1 = _sc_prop(xs1, eidx3, zfeat)
    xs2 = _tc_mid(acc1, xs1, dinv, W1, b1.reshape(1, 256), W2)
    acc2 = _sc_prop(xs2, eidx3, zfeat)
    Wgz = jnp.concatenate([Wxz[:DIN], Wxh[:DIN]], axis=1)
    Wgx = jnp.concatenate([Wxz[DIN:], Wxh[DIN:]], axis=1)
    bg = jnp.concatenate([bxz + bhz, bxh + bhh]).reshape(1, 512)
    H, out, _ = _tc_fin(
        acc2, xs2, dinv, x, b2.reshape(1, DIN), Wgz, Wgx, bg,
        Wo, bo.reshape(1, 1))
    return (out.reshape(1), H)
